# baseline (speedup 1.0000x reference)
import jax
import jax.numpy as jnp
from jax import lax
from jax.experimental import pallas as pl
from jax.experimental.pallas import tpu as pltpu
from jax.experimental.pallas import tpu_sc as plsc

_HIDDEN = 64
_L = 16
_NC = 2
_NS = 16
_NW = _NC * _NS
_CH = 128


def _emb_body(idx_hbm, tab_hbm, out_hbm, idx_v, rows_v, sem):
    nrows = idx_hbm.shape[0]
    bpw = nrows // _NW
    nchunk = bpw // _CH
    wid = lax.axis_index("s") * _NC + lax.axis_index("c")
    wbase = wid * bpw

    def chunk_body(t, carry):
        base = wbase + t * _CH
        pltpu.sync_copy(idx_hbm.at[pl.ds(base, _CH)], idx_v)
        pltpu.async_copy(tab_hbm.at[idx_v], rows_v, sem).wait()

        def grp_body(g, c):
            idxg = idx_v[pl.ds(g * _L, _L)]
            m16 = jnp.where(idxg != 0, jnp.float32(8.0), jnp.float32(0.0))
            for r in range(_L):
                row = g * _L + r
                mr = m16.at[jnp.full((_L,), r, jnp.int32)].get(
                    mode="promise_in_bounds")
                for k in range(_HIDDEN // _L):
                    sl = pl.ds(k * _L, _L)
                    rows_v[row, sl] = rows_v[row, sl] * mr
            return c

        lax.fori_loop(0, _CH // _L, grp_body, 0)
        pltpu.sync_copy(rows_v, out_hbm.at[pl.ds(base, _CH)])
        return carry

    lax.fori_loop(0, nchunk, chunk_body, 0)


@jax.jit
def kernel(inputs, weights):
    b, s = inputs.shape
    idx = inputs.reshape(-1).astype(jnp.int32)
    mesh = plsc.VectorSubcoreMesh(core_axis_name="c", subcore_axis_name="s")
    out = pl.kernel(
        _emb_body,
        out_type=jax.ShapeDtypeStruct((b * s, _HIDDEN), jnp.float32),
        mesh=mesh,
        scratch_types=[
            pltpu.VMEM((_CH,), jnp.int32),
            pltpu.VMEM((_CH, _HIDDEN), jnp.float32),
            pltpu.SemaphoreType.DMA,
        ],
        compiler_params=pltpu.CompilerParams(use_tc_tiling_on_sc=False),
    )(idx, weights)
    return out.reshape(b, s, _HIDDEN)


# speedup vs baseline: 1.1035x; 1.1035x over previous
import jax
import jax.numpy as jnp
from jax import lax
from jax.experimental import pallas as pl
from jax.experimental.pallas import tpu as pltpu
from jax.experimental.pallas import tpu_sc as plsc

_HIDDEN = 64
_L = 16
_NC = 2
_NS = 16
_NW = _NC * _NS
_CH = 128
_NBUF = 4


def _emb_body(idx_hbm, tab_hbm, out_hbm, idx_all, rows_v, gsem, osem):
    nchunk_rows, ch = idx_hbm.shape
    nchunk = nchunk_rows // _NW
    wid = lax.axis_index("s") * _NC + lax.axis_index("c")
    wrow = wid * nchunk

    pltpu.sync_copy(idx_hbm.at[pl.ds(wrow, nchunk)], idx_all)

    def gather_desc(t, b):
        return pltpu.make_async_copy(
            tab_hbm.at[idx_all.at[t]], rows_v.at[b], gsem.at[b])

    def out_desc(t, b):
        return pltpu.make_async_copy(
            rows_v.at[b], out_hbm.at[pl.ds((wrow + t) * ch, ch)], osem.at[b])

    gather_desc(0, 0).start()

    def chunk_body(t, carry):
        b = lax.rem(t, _NBUF)
        nb = lax.rem(t + 1, _NBUF)

        @pl.when(t + 1 < nchunk)
        def _():
            @pl.when(t + 1 >= _NBUF)
            def _():
                out_desc(t + 1 - _NBUF, nb).wait()
            gather_desc(t + 1, nb).start()

        gather_desc(t, b).wait()

        def grp_body(g, c):
            idxg = idx_all[t, pl.ds(g * _L, _L)]
            m16 = jnp.where(idxg != 0, jnp.float32(8.0), jnp.float32(0.0))
            for r in range(_L):
                row = g * _L + r
                mr = m16.at[jnp.full((_L,), r, jnp.int32)].get(
                    mode="promise_in_bounds")
                for k in range(_HIDDEN // _L):
                    sl = pl.ds(k * _L, _L)
                    rows_v[b, row, sl] = rows_v[b, row, sl] * mr
            return c

        lax.fori_loop(0, ch // _L, grp_body, 0)
        out_desc(t, b).start()
        return carry

    lax.fori_loop(0, nchunk, chunk_body, 0)

    def drain_body(k, carry):
        t = nchunk - 1 - k
        out_desc(t, lax.rem(t, _NBUF)).wait()
        return carry

    lax.fori_loop(0, min(_NBUF, nchunk), drain_body, 0)


@jax.jit
def kernel(inputs, weights):
    b, s = inputs.shape
    n = b * s
    idx = inputs.reshape(n // _CH, _CH).astype(jnp.int32)
    mesh = plsc.VectorSubcoreMesh(core_axis_name="c", subcore_axis_name="s")
    out = pl.kernel(
        _emb_body,
        out_type=jax.ShapeDtypeStruct((n, _HIDDEN), jnp.float32),
        mesh=mesh,
        scratch_types=[
            pltpu.VMEM((n // (_NW * _CH), _CH), jnp.int32),
            pltpu.VMEM((_NBUF, _CH, _HIDDEN), jnp.float32),
            pltpu.SemaphoreType.DMA((_NBUF,)),
            pltpu.SemaphoreType.DMA((_NBUF,)),
        ],
        compiler_params=pltpu.CompilerParams(use_tc_tiling_on_sc=False),
    )(idx, weights)
    return out.reshape(b, s, _HIDDEN)
